# Initial kernel scaffold; baseline (speedup 1.0000x reference)
import jax
import jax.numpy as jnp
from jax.experimental import pallas as pl


def kernel(x, table):
    raise NotImplementedError("write your pallas kernel here")


# speedup vs baseline: 4.8111x; 4.8111x over previous
import functools

import jax
import jax.numpy as jnp
from jax import lax
from jax.experimental import pallas as pl
from jax.experimental.pallas import tpu as pltpu
from jax.experimental.pallas import tpu_sc as plsc

EMB_DIM = 32
IDX_ROW = 128
ROWS_PER_CHUNK = 8
CHUNK = IDX_ROW * ROWS_PER_CHUNK
NBUF = 2


@functools.lru_cache(maxsize=None)
def _make_gather(num_idx, vocab):
    info = plsc.get_sparse_core_info()
    nc, ns = info.num_cores, info.num_subcores
    nw = nc * ns
    assert num_idx % (nw * CHUNK) == 0
    per_w = num_idx // nw
    rows_w = per_w // IDX_ROW
    chunks = per_w // CHUNK
    assert chunks % NBUF == 0

    mesh = plsc.VectorSubcoreMesh(core_axis_name="c", subcore_axis_name="s")

    @functools.partial(
        pl.kernel,
        out_type=jax.ShapeDtypeStruct((num_idx, EMB_DIM), jnp.float32),
        mesh=mesh,
        compiler_params=pltpu.CompilerParams(use_tc_tiling_on_sc=False),
        scratch_types=[
            pltpu.VMEM((NBUF, ROWS_PER_CHUNK, IDX_ROW), jnp.int32),
            pltpu.VMEM((NBUF, CHUNK, EMB_DIM), jnp.float32),
            pltpu.SemaphoreType.DMA,
        ],
    )
    def gather_kernel(x_hbm, table_hbm, out_hbm, idx_v, rows_v, gsem):
        wid = lax.axis_index("s") * nc + lax.axis_index("c")
        row_base = wid * rows_w
        out_base = wid * per_w

        def body(i, carry):
            for b in range(NBUF):
                g = i * NBUF + b
                roff = row_base + g * ROWS_PER_CHUNK
                ooff = out_base + g * CHUNK
                pltpu.sync_copy(x_hbm.at[pl.ds(roff, ROWS_PER_CHUNK)],
                                idx_v.at[b])
                cps = [
                    pltpu.async_copy(
                        table_hbm.at[idx_v.at[b, j]],
                        rows_v.at[b, pl.ds(j * IDX_ROW, IDX_ROW)],
                        gsem,
                    )
                    for j in range(ROWS_PER_CHUNK)
                ]
                for cp in cps:
                    cp.wait()
                pltpu.sync_copy(rows_v.at[b],
                                out_hbm.at[pl.ds(ooff, CHUNK)])
            return carry

        lax.fori_loop(0, chunks // NBUF, body, 0)

    return gather_kernel


def kernel(x, table):
    num_idx = x.size
    x_rows = x.reshape(num_idx // IDX_ROW, IDX_ROW)
    out = _make_gather(num_idx, table.shape[0])(x_rows, table)
    return out.reshape(x.shape + (EMB_DIM,))


# speedup vs baseline: 5.0530x; 1.0503x over previous
import functools

import jax
import jax.numpy as jnp
from jax import lax
from jax.experimental import pallas as pl
from jax.experimental.pallas import tpu as pltpu
from jax.experimental.pallas import tpu_sc as plsc

EMB_DIM = 32
IDX_ROW = 128
ROWS_PER_CHUNK = 8
CHUNK = IDX_ROW * ROWS_PER_CHUNK
NBUF = 3


@functools.lru_cache(maxsize=None)
def _make_gather(num_idx, vocab):
    info = plsc.get_sparse_core_info()
    nc, ns = info.num_cores, info.num_subcores
    nw = nc * ns
    assert num_idx % (nw * CHUNK) == 0
    per_w = num_idx // nw
    rows_w = per_w // IDX_ROW
    chunks = per_w // CHUNK
    assert chunks >= NBUF + 1 and (chunks - NBUF - 1) % NBUF == 0

    mesh = plsc.VectorSubcoreMesh(core_axis_name="c", subcore_axis_name="s")

    @functools.partial(
        pl.kernel,
        out_type=jax.ShapeDtypeStruct((num_idx, EMB_DIM), jnp.float32),
        mesh=mesh,
        compiler_params=pltpu.CompilerParams(use_tc_tiling_on_sc=False),
        scratch_types=[
            pltpu.VMEM((NBUF, ROWS_PER_CHUNK, IDX_ROW), jnp.int32),
            pltpu.VMEM((NBUF, CHUNK, EMB_DIM), jnp.float32),
            pltpu.SemaphoreType.DMA((NBUF,)),
            pltpu.SemaphoreType.DMA((NBUF,)),
            pltpu.SemaphoreType.DMA((NBUF,)),
        ],
    )
    def gather_kernel(x_hbm, table_hbm, out_hbm, idx_v, rows_v,
                      isem, gsem, wsem):
        wid = lax.axis_index("s") * nc + lax.axis_index("c")
        row_base = wid * rows_w
        out_base = wid * per_w
        last = chunks - 1

        def idx_copy(g, s):
            return pltpu.make_async_copy(
                x_hbm.at[pl.ds(row_base + g * ROWS_PER_CHUNK, ROWS_PER_CHUNK)],
                idx_v.at[s], isem.at[s])

        def gath_copy(s, j):
            return pltpu.make_async_copy(
                table_hbm.at[idx_v.at[s, j]],
                rows_v.at[s, pl.ds(j * IDX_ROW, IDX_ROW)], gsem.at[s])

        def wb_copy(g, s):
            return pltpu.make_async_copy(
                rows_v.at[s], out_hbm.at[pl.ds(out_base + g * CHUNK, CHUNK)],
                wsem.at[s])

        def fire_gathers(s):
            for j in range(ROWS_PER_CHUNK):
                gath_copy(s, j).start()

        def drain_gathers(s):
            for j in range(ROWS_PER_CHUNK):
                gath_copy(s, j).wait()

        idx_copy(0, 0).start()
        idx_copy(1, 1).start()
        idx_copy(0, 0).wait()
        fire_gathers(0)
        idx_copy(2, 2).start()
        idx_copy(1, 1).wait()
        fire_gathers(1)
        drain_gathers(0)
        wb_copy(0, 0).start()
        idx_copy(3, 0).start()
        idx_copy(2, 2).wait()
        fire_gathers(2)
        drain_gathers(1)
        wb_copy(1, 1).start()
        idx_copy(4, 1).start()

        def body(k, carry):
            for b in range(NBUF):
                g = NBUF + k * NBUF + b
                sp = (b + NBUF - 1) % NBUF
                idx_copy(g, b).wait()
                wb_copy(g - NBUF, b).wait()
                fire_gathers(b)
                drain_gathers(sp)
                wb_copy(g - 1, sp).start()
                idx_copy(jnp.minimum(g + 2, last), sp).start()
            return carry

        lax.fori_loop(0, (chunks - NBUF - 1) // NBUF, body, 0)

        g = last
        s = last % NBUF
        sp = (s + NBUF - 1) % NBUF
        idx_copy(g, s).wait()
        wb_copy(g - NBUF, s).wait()
        fire_gathers(s)
        drain_gathers(sp)
        wb_copy(g - 1, sp).start()
        drain_gathers(s)
        wb_copy(g, s).start()
        wb_copy(g - 2, (s + 1) % NBUF).wait()
        wb_copy(g - 1, sp).wait()
        wb_copy(g, s).wait()
        idx_copy(last, (s + 1) % NBUF).wait()

    return gather_kernel


def kernel(x, table):
    num_idx = x.size
    x_rows = x.reshape(num_idx // IDX_ROW, IDX_ROW)
    out = _make_gather(num_idx, table.shape[0])(x_rows, table)
    return out.reshape(x.shape + (EMB_DIM,))
